# baseline (speedup 1.0000x reference)
import functools

import jax
import jax.numpy as jnp
import numpy as np
from jax.experimental import pallas as pl
from jax.experimental.pallas import tpu as pltpu

_IGNORE = 255
_THRESH = float(-np.log(0.7))

_B, _C, _H, _W = 8, 19, 512, 512
_NPIX = _B * _H * _W
_NMIN = _NPIX // 16
_L = 4096


def _loss_body(preds_ref, labels_ref, loss_ref, cnt_ref, sum_ref, acc_ref):
    i = pl.program_id(0)
    j = pl.program_id(1)

    @pl.when((i == 0) & (j == 0))
    def _init():
        acc_ref[0] = 0.0
        acc_ref[1] = 0.0

    x = preds_ref[0]
    lab = labels_ref[0]
    m = jnp.max(x, axis=0, keepdims=True)
    sh = x - m
    s = jnp.sum(jnp.exp(sh), axis=0, keepdims=True)
    logs = jnp.log(s)
    ch = jax.lax.broadcasted_iota(jnp.int32, (_C, _L), 0)
    picked = jnp.sum(jnp.where(ch == lab, sh, 0.0), axis=0, keepdims=True)
    nll = logs - picked
    valid = lab != _IGNORE
    loss = jnp.where(valid, nll, 0.0)
    loss_ref[0] = loss

    hard = loss > _THRESH
    acc_ref[0] += jnp.sum(jnp.where(hard, 1.0, 0.0))
    acc_ref[1] += jnp.sum(jnp.where(hard, loss, 0.0))

    @pl.when((i == _B - 1) & (j == (_H * _W) // _L - 1))
    def _fin():
        cnt_ref[0, 0] = acc_ref[0]
        sum_ref[0, 0] = acc_ref[1]


@functools.partial(jax.jit, static_argnames=("interpret",))
def _loss_call(preds, labels, interpret=False):
    pr = preds.reshape(_B, _C, _H * _W)
    lb = labels.reshape(_B, 1, _H * _W)
    grid = (_B, (_H * _W) // _L)
    loss, cnt, hsum = pl.pallas_call(
        _loss_body,
        grid=grid,
        in_specs=[
            pl.BlockSpec((1, _C, _L), lambda i, j: (i, 0, j)),
            pl.BlockSpec((1, 1, _L), lambda i, j: (i, 0, j)),
        ],
        out_specs=[
            pl.BlockSpec((1, 1, _L), lambda i, j: (i, 0, j)),
            pl.BlockSpec(memory_space=pltpu.SMEM),
            pl.BlockSpec(memory_space=pltpu.SMEM),
        ],
        out_shape=[
            jax.ShapeDtypeStruct((_B, 1, _H * _W), jnp.float32),
            jax.ShapeDtypeStruct((1, 1), jnp.float32),
            jax.ShapeDtypeStruct((1, 1), jnp.float32),
        ],
        scratch_shapes=[pltpu.SMEM((2,), jnp.float32)],
        interpret=interpret,
    )(pr, lb)
    return loss.reshape(-1), cnt[0, 0], hsum[0, 0]


def kernel(preds, labels):
    loss, n_hard, hard_sum = _loss_call(preds, labels)
    topk, _ = jax.lax.top_k(loss, _NMIN)
    mean_topk = jnp.mean(topk)
    mean_hard = hard_sum / n_hard
    return jnp.where(n_hard < jnp.float32(_NMIN), mean_topk, mean_hard)


# speedup vs baseline: 3.1539x; 3.1539x over previous
import functools

import jax
import jax.numpy as jnp
import numpy as np
from jax import lax
from jax.experimental import pallas as pl
from jax.experimental.pallas import tpu as pltpu
from jax.experimental.pallas import tpu_sc as plsc

_IGNORE = 255
_THRESH = float(-np.log(0.7))

_B, _C, _H, _W = 8, 19, 512, 512
_NPIX = _B * _H * _W
_NMIN = _NPIX // 16
_L = 4096


def _loss_body(preds_ref, labels_ref, loss_ref, cnt_ref, sum_ref, acc_ref):
    i = pl.program_id(0)
    j = pl.program_id(1)

    @pl.when((i == 0) & (j == 0))
    def _init():
        acc_ref[0] = 0.0
        acc_ref[1] = 0.0

    x = preds_ref[0]
    lab = labels_ref[0]
    m = jnp.max(x, axis=0, keepdims=True)
    sh = x - m
    s = jnp.sum(jnp.exp(sh), axis=0, keepdims=True)
    logs = jnp.log(s)
    ch = jax.lax.broadcasted_iota(jnp.int32, (_C, _L), 0)
    picked = jnp.sum(jnp.where(ch == lab, sh, 0.0), axis=0, keepdims=True)
    nll = logs - picked
    valid = lab != _IGNORE
    loss = jnp.where(valid, nll, 0.0)
    loss_ref[0] = loss

    hard = loss > _THRESH
    acc_ref[0] += jnp.sum(jnp.where(hard, 1.0, 0.0))
    acc_ref[1] += jnp.sum(jnp.where(hard, loss, 0.0))

    @pl.when((i == _B - 1) & (j == (_H * _W) // _L - 1))
    def _fin():
        cnt_ref[0, 0] = acc_ref[0]
        sum_ref[0, 0] = acc_ref[1]


@functools.partial(jax.jit, static_argnames=("interpret",))
def _loss_call(preds, labels, interpret=False):
    pr = preds.reshape(_B, _C, _H * _W)
    lb = labels.reshape(_B, 1, _H * _W)
    grid = (_B, (_H * _W) // _L)
    loss, cnt, hsum = pl.pallas_call(
        _loss_body,
        grid=grid,
        in_specs=[
            pl.BlockSpec((1, _C, _L), lambda i, j: (i, 0, j)),
            pl.BlockSpec((1, 1, _L), lambda i, j: (i, 0, j)),
        ],
        out_specs=[
            pl.BlockSpec((1, 1, _L), lambda i, j: (i, 0, j)),
            pl.BlockSpec(memory_space=pltpu.SMEM),
            pl.BlockSpec(memory_space=pltpu.SMEM),
        ],
        out_shape=[
            jax.ShapeDtypeStruct((_B, 1, _H * _W), jnp.float32),
            jax.ShapeDtypeStruct((1, 1), jnp.float32),
            jax.ShapeDtypeStruct((1, 1), jnp.float32),
        ],
        scratch_shapes=[pltpu.SMEM((2,), jnp.float32)],
        interpret=interpret,
    )(pr, lb)
    return loss.reshape(-1), cnt[0, 0], hsum[0, 0]


_NT = 16
_PT = _NPIX // _NT
_CH = 8192
_NCH = _PT // _CH
_NB = 2048
_NG = _NB // 16


def _sc_topk_body(loss_hbm, out_hbm, buf, hist_c, hist_s, m1, m2c, m2s, tmp_c,
                  tmp_s, res, sh1, sh2c, sh2s):
    cid = lax.axis_index("c")
    sid = lax.axis_index("s")
    lane = lax.iota(jnp.int32, 16)
    base = sid * _PT
    ones_i = jnp.ones((16,), jnp.int32)
    zeros_i = jnp.zeros((16,), jnp.int32)
    zeros_f = jnp.zeros((16,), jnp.float32)

    def _zero_hist(ref, zval):
        def zb(g, _):
            ref[pl.ds(g * 16, 16)] = zval
            return 0
        lax.fori_loop(0, _NB, zb, 0)

    _zero_hist(hist_c, zeros_i)

    def chunk1(c, _):
        pltpu.sync_copy(loss_hbm.at[pl.ds(base + c * _CH, _CH)], buf)

        def inner(v, _):
            x = buf[pl.ds(v * 16, 16)]
            k = lax.bitcast_convert_type(x, jnp.int32) & jnp.int32(0x7FFFFFFF)
            b1 = lax.shift_right_logical(k, 20)
            plsc.addupdate_scatter(hist_c, [b1 * 16 + lane], ones_i)
            return 0
        lax.fori_loop(0, _CH // 16, inner, 0)
        return 0
    lax.fori_loop(0, _NCH, chunk1, 0)

    def lm1(g, _):
        acc = zeros_i
        for c in range(16):
            acc = acc + plsc.load_gather(hist_c, [g * 256 + lane * 16 + c])
        m1[pl.ds(g * 16, 16)] = acc
        return 0
    lax.fori_loop(0, _NG, lm1, 0)

    pltpu.sync_copy(m1, sh1.at[sid])
    plsc.subcore_barrier()
    for t in range(_NT):
        pltpu.sync_copy(sh1.at[t], tmp_c)
        if t == 0:
            def cp0(g, _):
                m1[pl.ds(g * 16, 16)] = tmp_c[pl.ds(g * 16, 16)]
                return 0
            lax.fori_loop(0, _NG, cp0, 0)
        else:
            def acc1(g, _):
                sl = pl.ds(g * 16, 16)
                m1[sl] = m1[sl] + tmp_c[sl]
                return 0
            lax.fori_loop(0, _NG, acc1, 0)

    T1 = jnp.int32(_NPIX - _NMIN)

    def sc1(g, carry):
        run, cnt = carry
        v = m1[pl.ds(g * 16, 16)]
        cs = plsc.cumsum(v)
        pe = cs + run - v
        pc = plsc.all_reduce_population_count(pe <= T1)
        return run + jnp.max(cs), cnt + jnp.max(pc)
    _, cnt1 = lax.fori_loop(0, _NG, sc1, (jnp.int32(0), jnp.int32(0)))
    b1s = cnt1 - 1
    b1v = jnp.broadcast_to(b1s, (16,))

    def ca(g, acc):
        v = m1[pl.ds(g * 16, 16)]
        binid = g * 16 + lane
        return acc + jnp.sum(jnp.where(binid > b1v, v, zeros_i))
    c_above = lax.fori_loop(0, _NG, ca, jnp.int32(0))

    _zero_hist(hist_c, zeros_i)

    def zs(g, _):
        hist_s[pl.ds(g * 16, 16)] = zeros_f
        return 0
    lax.fori_loop(0, _NB + 1, zs, 0)

    def chunk2(c, _):
        pltpu.sync_copy(loss_hbm.at[pl.ds(base + c * _CH, _CH)], buf)

        def inner(v, _):
            x = buf[pl.ds(v * 16, 16)]
            k = lax.bitcast_convert_type(x, jnp.int32) & jnp.int32(0x7FFFFFFF)
            b1 = lax.shift_right_logical(k, 20)
            eq = b1 == b1v
            gt = b1 > b1v
            b2 = lax.shift_right_logical(k, 9) & jnp.int32(0x7FF)
            idx = b2 * 16 + lane
            sidx = jnp.where(eq, idx, jnp.int32(_NB * 16) + lane)
            plsc.addupdate_scatter(hist_c, [idx], ones_i, mask=eq)
            plsc.addupdate_scatter(hist_s, [sidx], x, mask=eq | gt)
            return 0
        lax.fori_loop(0, _CH // 16, inner, 0)
        return 0
    lax.fori_loop(0, _NCH, chunk2, 0)

    def lm2(g, _):
        accc = zeros_i
        accs = zeros_f
        for c in range(16):
            gi = g * 256 + lane * 16 + c
            accc = accc + plsc.load_gather(hist_c, [gi])
            accs = accs + plsc.load_gather(hist_s, [gi])
        m2c[pl.ds(g * 16, 16)] = accc
        m2s[pl.ds(g * 16, 16)] = accs
        return 0
    lax.fori_loop(0, _NG, lm2, 0)
    m2s[pl.ds(_NB, 16)] = hist_s[pl.ds(_NB * 16, 16)]

    pltpu.sync_copy(m2c, sh2c.at[sid])
    pltpu.sync_copy(m2s, sh2s.at[sid])
    plsc.subcore_barrier()
    for t in range(_NT):
        pltpu.sync_copy(sh2c.at[t], tmp_c)
        pltpu.sync_copy(sh2s.at[t], tmp_s)
        if t == 0:
            def cp2(g, _):
                sl = pl.ds(g * 16, 16)
                m2c[sl] = tmp_c[sl]
                return 0
            lax.fori_loop(0, _NG, cp2, 0)

            def cp2s(g, _):
                sl = pl.ds(g * 16, 16)
                m2s[sl] = tmp_s[sl]
                return 0
            lax.fori_loop(0, _NG + 1, cp2s, 0)
        else:
            def acc2(g, _):
                sl = pl.ds(g * 16, 16)
                m2c[sl] = m2c[sl] + tmp_c[sl]
                return 0
            lax.fori_loop(0, _NG, acc2, 0)

            def acc2s(g, _):
                sl = pl.ds(g * 16, 16)
                m2s[sl] = m2s[sl] + tmp_s[sl]
                return 0
            lax.fori_loop(0, _NG + 1, acc2s, 0)
    s_above = jnp.sum(m2s[pl.ds(_NB, 16)])

    total2 = jnp.max(plsc.load_gather(m1, [b1v]))
    n2 = jnp.int32(_NMIN) - c_above
    T2 = total2 - n2

    def sc2(g, carry):
        run, cnt = carry
        v = m2c[pl.ds(g * 16, 16)]
        cs = plsc.cumsum(v)
        pe = cs + run - v
        pc = plsc.all_reduce_population_count(pe <= T2)
        return run + jnp.max(cs), cnt + jnp.max(pc)
    _, cnt2 = lax.fori_loop(0, _NG, sc2, (jnp.int32(0), jnp.int32(0)))
    b2s = cnt2 - 1
    b2v = jnp.broadcast_to(b2s, (16,))

    def suf2(g, carry):
        accc, accs = carry
        binid = g * 16 + lane
        gt = binid > b2v
        accc = accc + jnp.sum(jnp.where(gt, m2c[pl.ds(g * 16, 16)], zeros_i))
        accs = accs + jnp.sum(jnp.where(gt, m2s[pl.ds(g * 16, 16)], zeros_f))
        return accc, accs
    c_hi2, s_hi2 = lax.fori_loop(0, _NG, suf2, (jnp.int32(0), jnp.float32(0.0)))

    c_hi = c_above + c_hi2
    s_hi = s_above + s_hi2
    tbits = jnp.broadcast_to(
        lax.shift_left(b1s, 20) | lax.shift_left(b2s, 9) | jnp.int32(256), (16,))
    t_rep = lax.bitcast_convert_type(tbits, jnp.float32)
    rem = (jnp.int32(_NMIN) - c_hi).astype(jnp.float32)
    res[...] = (s_hi + rem * t_rep) * jnp.float32(1.0 / _NMIN)

    @pl.when((cid == 0) & (sid == 0))
    def _out():
        pltpu.sync_copy(res, out_hbm)


@jax.jit
def _sc_topk_call(loss):
    mesh = plsc.VectorSubcoreMesh(core_axis_name="c", subcore_axis_name="s")
    f = functools.partial(
        pl.kernel,
        out_type=jax.ShapeDtypeStruct((16,), jnp.float32),
        mesh=mesh,
        compiler_params=pltpu.CompilerParams(needs_layout_passes=False),
        scratch_types=[
            pltpu.VMEM((_CH,), jnp.float32),
            pltpu.VMEM((_NB * 16 + 16,), jnp.int32),
            pltpu.VMEM((_NB * 16 + 16,), jnp.float32),
            pltpu.VMEM((_NB,), jnp.int32),
            pltpu.VMEM((_NB,), jnp.int32),
            pltpu.VMEM((_NB + 16,), jnp.float32),
            pltpu.VMEM((_NB,), jnp.int32),
            pltpu.VMEM((_NB + 16,), jnp.float32),
            pltpu.VMEM((16,), jnp.float32),
            pltpu.VMEM_SHARED((_NT, _NB), jnp.int32),
            pltpu.VMEM_SHARED((_NT, _NB), jnp.int32),
            pltpu.VMEM_SHARED((_NT, _NB + 16), jnp.float32),
        ],
    )(_sc_topk_body)
    return f(loss)


def kernel(preds, labels):
    loss, n_hard, hard_sum = _loss_call(preds, labels)
    mean_topk = _sc_topk_call(loss)[0]
    mean_hard = hard_sum / n_hard
    return jnp.where(n_hard < jnp.float32(_NMIN), mean_topk, mean_hard)


# speedup vs baseline: 6.0269x; 1.9109x over previous
import functools

import jax
import jax.numpy as jnp
import numpy as np
from jax import lax
from jax.experimental import pallas as pl
from jax.experimental.pallas import tpu as pltpu
from jax.experimental.pallas import tpu_sc as plsc

_IGNORE = 255
_THRESH = float(-np.log(0.7))

_B, _C, _H, _W = 8, 19, 512, 512
_NPIX = _B * _H * _W
_NMIN = _NPIX // 16
_L = 4096


_BH = 16


def _loss_body(preds_ref, labels_ref, loss_ref, cnt_ref, sum_ref, acc_ref):
    i = pl.program_id(0)
    j = pl.program_id(1)

    @pl.when((i == 0) & (j == 0))
    def _init():
        acc_ref[0] = 0.0
        acc_ref[1] = 0.0

    x = preds_ref[0]
    lab = labels_ref[0]
    m = jnp.max(x, axis=0)
    sh = x - m[None]
    s = jnp.sum(jnp.exp(sh), axis=0)
    logs = jnp.log(s)
    ch = jax.lax.broadcasted_iota(jnp.int32, (_C, _BH, _W), 0)
    picked = jnp.sum(jnp.where(ch == lab[None], sh, 0.0), axis=0)
    nll = logs - picked
    valid = lab != _IGNORE
    loss = jnp.where(valid, nll, 0.0)
    loss_ref[0] = loss

    hard = loss > _THRESH
    acc_ref[0] += jnp.sum(jnp.where(hard, 1.0, 0.0))
    acc_ref[1] += jnp.sum(jnp.where(hard, loss, 0.0))

    @pl.when((i == _B - 1) & (j == _H // _BH - 1))
    def _fin():
        cnt_ref[0, 0] = acc_ref[0]
        sum_ref[0, 0] = acc_ref[1]


@functools.partial(jax.jit, static_argnames=("interpret",))
def _loss_call(preds, labels, interpret=False):
    grid = (_B, _H // _BH)
    loss, cnt, hsum = pl.pallas_call(
        _loss_body,
        grid=grid,
        in_specs=[
            pl.BlockSpec((1, _C, _BH, _W), lambda i, j: (i, 0, j, 0)),
            pl.BlockSpec((1, _BH, _W), lambda i, j: (i, j, 0)),
        ],
        out_specs=[
            pl.BlockSpec((1, _BH, _W), lambda i, j: (i, j, 0)),
            pl.BlockSpec(memory_space=pltpu.SMEM),
            pl.BlockSpec(memory_space=pltpu.SMEM),
        ],
        out_shape=[
            jax.ShapeDtypeStruct((_B, _H, _W), jnp.float32),
            jax.ShapeDtypeStruct((1, 1), jnp.float32),
            jax.ShapeDtypeStruct((1, 1), jnp.float32),
        ],
        scratch_shapes=[pltpu.SMEM((2,), jnp.float32)],
        interpret=interpret,
    )(preds, labels)
    return loss.reshape(-1), cnt[0, 0], hsum[0, 0]


_NT = 16
_PT = _NPIX // _NT
_CH = 8192
_NCH = _PT // _CH
_NB = 2048
_NG = _NB // 16


def _sc_topk_body(loss_hbm, out_hbm, buf, hist_c, hist_s, m1, m2c, m2s, tmp_c,
                  tmp_s, res, sh1, sh2c, sh2s):
    cid = lax.axis_index("c")
    sid = lax.axis_index("s")
    lane = lax.iota(jnp.int32, 16)
    base = sid * _PT
    ones_i = jnp.ones((16,), jnp.int32)
    zeros_i = jnp.zeros((16,), jnp.int32)
    zeros_f = jnp.zeros((16,), jnp.float32)

    def _zero_hist(ref, zval):
        def zb(g, _):
            ref[pl.ds(g * 16, 16)] = zval
            return 0
        lax.fori_loop(0, _NB, zb, 0, unroll=8)

    _zero_hist(hist_c, zeros_i)

    def chunk1(c, _):
        pltpu.sync_copy(loss_hbm.at[pl.ds(base + c * _CH, _CH)], buf)

        def inner(v, _):
            x = buf[pl.ds(v * 16, 16)]
            k = lax.bitcast_convert_type(x, jnp.int32) & jnp.int32(0x7FFFFFFF)
            b1 = lax.shift_right_logical(k, 20)
            plsc.addupdate_scatter(hist_c, [b1 * 16 + lane], ones_i)
            return 0
        lax.fori_loop(0, _CH // 16, inner, 0, unroll=8)
        return 0
    lax.fori_loop(0, _NCH, chunk1, 0)

    def lm1(g, _):
        acc = zeros_i
        for c in range(16):
            acc = acc + plsc.load_gather(hist_c, [g * 256 + lane * 16 + c])
        m1[pl.ds(g * 16, 16)] = acc
        return 0
    lax.fori_loop(0, _NG, lm1, 0, unroll=2)

    pltpu.sync_copy(m1, sh1.at[sid])
    plsc.subcore_barrier()
    for t in range(_NT):
        pltpu.sync_copy(sh1.at[t], tmp_c)
        if t == 0:
            def cp0(g, _):
                m1[pl.ds(g * 16, 16)] = tmp_c[pl.ds(g * 16, 16)]
                return 0
            lax.fori_loop(0, _NG, cp0, 0, unroll=4)
        else:
            def acc1(g, _):
                sl = pl.ds(g * 16, 16)
                m1[sl] = m1[sl] + tmp_c[sl]
                return 0
            lax.fori_loop(0, _NG, acc1, 0, unroll=4)

    T1 = jnp.int32(_NPIX - _NMIN)

    def sc1(g, carry):
        run, cnt = carry
        v = m1[pl.ds(g * 16, 16)]
        cs = plsc.cumsum(v)
        pe = cs + run - v
        pc = plsc.all_reduce_population_count(pe <= T1)
        return run + jnp.max(cs), cnt + jnp.max(pc)
    _, cnt1 = lax.fori_loop(0, _NG, sc1, (jnp.int32(0), jnp.int32(0)))
    b1s = cnt1 - 1
    b1v = jnp.broadcast_to(b1s, (16,))

    def ca(g, acc):
        v = m1[pl.ds(g * 16, 16)]
        binid = g * 16 + lane
        return acc + jnp.sum(jnp.where(binid > b1v, v, zeros_i))
    c_above = lax.fori_loop(0, _NG, ca, jnp.int32(0))

    _zero_hist(hist_c, zeros_i)

    def zs(g, _):
        hist_s[pl.ds(g * 16, 16)] = zeros_f
        return 0
    lax.fori_loop(0, _NB + 1, zs, 0, unroll=8)

    def chunk2(c, _):
        pltpu.sync_copy(loss_hbm.at[pl.ds(base + c * _CH, _CH)], buf)

        def inner(v, _):
            x = buf[pl.ds(v * 16, 16)]
            k = lax.bitcast_convert_type(x, jnp.int32) & jnp.int32(0x7FFFFFFF)
            b1 = lax.shift_right_logical(k, 20)
            eq = b1 == b1v
            gt = b1 > b1v
            b2 = lax.shift_right_logical(k, 9) & jnp.int32(0x7FF)
            idx = b2 * 16 + lane
            sidx = jnp.where(eq, idx, jnp.int32(_NB * 16) + lane)
            plsc.addupdate_scatter(hist_c, [idx], ones_i, mask=eq)
            plsc.addupdate_scatter(hist_s, [sidx], x, mask=eq | gt)
            return 0
        lax.fori_loop(0, _CH // 16, inner, 0, unroll=8)
        return 0
    lax.fori_loop(0, _NCH, chunk2, 0)

    def lm2(g, _):
        accc = zeros_i
        accs = zeros_f
        for c in range(16):
            gi = g * 256 + lane * 16 + c
            accc = accc + plsc.load_gather(hist_c, [gi])
            accs = accs + plsc.load_gather(hist_s, [gi])
        m2c[pl.ds(g * 16, 16)] = accc
        m2s[pl.ds(g * 16, 16)] = accs
        return 0
    lax.fori_loop(0, _NG, lm2, 0, unroll=2)
    m2s[pl.ds(_NB, 16)] = hist_s[pl.ds(_NB * 16, 16)]

    pltpu.sync_copy(m2c, sh2c.at[sid])
    pltpu.sync_copy(m2s, sh2s.at[sid])
    plsc.subcore_barrier()
    for t in range(_NT):
        pltpu.sync_copy(sh2c.at[t], tmp_c)
        pltpu.sync_copy(sh2s.at[t], tmp_s)
        if t == 0:
            def cp2(g, _):
                sl = pl.ds(g * 16, 16)
                m2c[sl] = tmp_c[sl]
                return 0
            lax.fori_loop(0, _NG, cp2, 0, unroll=4)

            def cp2s(g, _):
                sl = pl.ds(g * 16, 16)
                m2s[sl] = tmp_s[sl]
                return 0
            lax.fori_loop(0, _NG + 1, cp2s, 0, unroll=4)
        else:
            def acc2(g, _):
                sl = pl.ds(g * 16, 16)
                m2c[sl] = m2c[sl] + tmp_c[sl]
                return 0
            lax.fori_loop(0, _NG, acc2, 0, unroll=4)

            def acc2s(g, _):
                sl = pl.ds(g * 16, 16)
                m2s[sl] = m2s[sl] + tmp_s[sl]
                return 0
            lax.fori_loop(0, _NG + 1, acc2s, 0, unroll=4)
    s_above = jnp.sum(m2s[pl.ds(_NB, 16)])

    total2 = jnp.max(plsc.load_gather(m1, [b1v]))
    n2 = jnp.int32(_NMIN) - c_above
    T2 = total2 - n2

    def sc2(g, carry):
        run, cnt = carry
        v = m2c[pl.ds(g * 16, 16)]
        cs = plsc.cumsum(v)
        pe = cs + run - v
        pc = plsc.all_reduce_population_count(pe <= T2)
        return run + jnp.max(cs), cnt + jnp.max(pc)
    _, cnt2 = lax.fori_loop(0, _NG, sc2, (jnp.int32(0), jnp.int32(0)))
    b2s = cnt2 - 1
    b2v = jnp.broadcast_to(b2s, (16,))

    def suf2(g, carry):
        accc, accs = carry
        binid = g * 16 + lane
        gt = binid > b2v
        accc = accc + jnp.sum(jnp.where(gt, m2c[pl.ds(g * 16, 16)], zeros_i))
        accs = accs + jnp.sum(jnp.where(gt, m2s[pl.ds(g * 16, 16)], zeros_f))
        return accc, accs
    c_hi2, s_hi2 = lax.fori_loop(0, _NG, suf2, (jnp.int32(0), jnp.float32(0.0)))

    c_hi = c_above + c_hi2
    s_hi = s_above + s_hi2
    tbits = jnp.broadcast_to(
        lax.shift_left(b1s, 20) | lax.shift_left(b2s, 9) | jnp.int32(256), (16,))
    t_rep = lax.bitcast_convert_type(tbits, jnp.float32)
    rem = (jnp.int32(_NMIN) - c_hi).astype(jnp.float32)
    res[...] = (s_hi + rem * t_rep) * jnp.float32(1.0 / _NMIN)

    @pl.when((cid == 0) & (sid == 0))
    def _out():
        pltpu.sync_copy(res, out_hbm)


@jax.jit
def _sc_topk_call(loss):
    mesh = plsc.VectorSubcoreMesh(core_axis_name="c", subcore_axis_name="s")
    f = functools.partial(
        pl.kernel,
        out_type=jax.ShapeDtypeStruct((16,), jnp.float32),
        mesh=mesh,
        compiler_params=pltpu.CompilerParams(needs_layout_passes=False),
        scratch_types=[
            pltpu.VMEM((_CH,), jnp.float32),
            pltpu.VMEM((_NB * 16 + 16,), jnp.int32),
            pltpu.VMEM((_NB * 16 + 16,), jnp.float32),
            pltpu.VMEM((_NB,), jnp.int32),
            pltpu.VMEM((_NB,), jnp.int32),
            pltpu.VMEM((_NB + 16,), jnp.float32),
            pltpu.VMEM((_NB,), jnp.int32),
            pltpu.VMEM((_NB + 16,), jnp.float32),
            pltpu.VMEM((16,), jnp.float32),
            pltpu.VMEM_SHARED((_NT, _NB), jnp.int32),
            pltpu.VMEM_SHARED((_NT, _NB), jnp.int32),
            pltpu.VMEM_SHARED((_NT, _NB + 16), jnp.float32),
        ],
    )(_sc_topk_body)
    return f(loss)


def kernel(preds, labels):
    loss, n_hard, hard_sum = _loss_call(preds, labels)
    mean_topk = _sc_topk_call(loss)[0]
    mean_hard = hard_sum / n_hard
    return jnp.where(n_hard < jnp.float32(_NMIN), mean_topk, mean_hard)


# speedup vs baseline: 6.0634x; 1.0061x over previous
import functools

import jax
import jax.numpy as jnp
import numpy as np
from jax import lax
from jax.experimental import pallas as pl
from jax.experimental.pallas import tpu as pltpu
from jax.experimental.pallas import tpu_sc as plsc

_IGNORE = 255
_THRESH = float(-np.log(0.7))

_B, _C, _H, _W = 8, 19, 512, 512
_NPIX = _B * _H * _W
_NMIN = _NPIX // 16
_L = 4096


_BH = 16


def _loss_body(preds_ref, labels_ref, loss_ref, cnt_ref, sum_ref, acc_ref):
    i = pl.program_id(0)
    j = pl.program_id(1)

    @pl.when((i == 0) & (j == 0))
    def _init():
        acc_ref[0] = 0.0
        acc_ref[1] = 0.0

    x = preds_ref[0]
    lab = labels_ref[0]
    m = jnp.max(x, axis=0)
    sh = x - m[None]
    s = jnp.sum(jnp.exp(sh), axis=0)
    logs = jnp.log(s)
    ch = jax.lax.broadcasted_iota(jnp.int32, (_C, _BH, _W), 0)
    picked = jnp.sum(jnp.where(ch == lab[None], sh, 0.0), axis=0)
    nll = logs - picked
    valid = lab != _IGNORE
    loss = jnp.where(valid, nll, 0.0)
    loss_ref[0] = loss

    hard = loss > _THRESH
    acc_ref[0] += jnp.sum(jnp.where(hard, 1.0, 0.0))
    acc_ref[1] += jnp.sum(jnp.where(hard, loss, 0.0))

    @pl.when((i == _B - 1) & (j == _H // _BH - 1))
    def _fin():
        cnt_ref[0, 0] = acc_ref[0]
        sum_ref[0, 0] = acc_ref[1]


@functools.partial(jax.jit, static_argnames=("interpret",))
def _loss_call(preds, labels, interpret=False):
    grid = (_B, _H // _BH)
    loss, cnt, hsum = pl.pallas_call(
        _loss_body,
        grid=grid,
        in_specs=[
            pl.BlockSpec((1, _C, _BH, _W), lambda i, j: (i, 0, j, 0)),
            pl.BlockSpec((1, _BH, _W), lambda i, j: (i, j, 0)),
        ],
        out_specs=[
            pl.BlockSpec((1, _BH, _W), lambda i, j: (i, j, 0)),
            pl.BlockSpec(memory_space=pltpu.SMEM),
            pl.BlockSpec(memory_space=pltpu.SMEM),
        ],
        out_shape=[
            jax.ShapeDtypeStruct((_B, _H, _W), jnp.float32),
            jax.ShapeDtypeStruct((1, 1), jnp.float32),
            jax.ShapeDtypeStruct((1, 1), jnp.float32),
        ],
        scratch_shapes=[pltpu.SMEM((2,), jnp.float32)],
        interpret=interpret,
    )(preds, labels)
    return loss.reshape(-1), cnt[0, 0], hsum[0, 0]


_NT = 16
_PT = _NPIX // _NT
_CH = 8192
_NCH = _PT // _CH
_NB = 2048
_NG = _NB // 16


def _sc_topk_body(loss_hbm, out_hbm, buf, hist_c, hist_s, m1, m2c, m2s, tmp_c,
                  tmp_s, res, sh1, sh2c, sh2s):
    cid = lax.axis_index("c")
    sid = lax.axis_index("s")
    lane = lax.iota(jnp.int32, 16)
    base = sid * _PT
    ones_i = jnp.ones((16,), jnp.int32)
    zeros_i = jnp.zeros((16,), jnp.int32)
    zeros_f = jnp.zeros((16,), jnp.float32)

    def _zero_hist(ref, zval):
        def zb(g, _):
            ref[pl.ds(g * 16, 16)] = zval
            return 0
        lax.fori_loop(0, _NB, zb, 0, unroll=8)

    _zero_hist(hist_c, zeros_i)
    _zero_hist(hist_s, zeros_f)
    ones_f = jnp.ones((16,), jnp.float32)

    def chunk1(c, _):
        pltpu.sync_copy(loss_hbm.at[pl.ds(base + c * _CH, _CH)], buf)

        def inner(v, _):
            for u in range(8):
                x = buf[pl.ds((v * 8 + u) * 16, 16)]
                k = lax.bitcast_convert_type(x, jnp.int32) & jnp.int32(0x7FFFFFFF)
                b1 = lax.shift_right_logical(k, 20)
                plsc.addupdate_scatter(hist_c, [b1 * 16 + lane], ones_i)
            return 0
        lax.fori_loop(0, _CH // 128, inner, 0)
        return 0
    lax.fori_loop(0, _NCH, chunk1, 0)

    def lm1(g, _):
        acc = zeros_i
        accf = zeros_f
        for c in range(16):
            acc = acc + plsc.load_gather(hist_c, [g * 256 + lane * 16 + c])
            accf = accf + plsc.load_gather(hist_s, [g * 256 + lane * 16 + c])
        m1[pl.ds(g * 16, 16)] = acc + accf.astype(jnp.int32)
        return 0
    lax.fori_loop(0, _NG, lm1, 0, unroll=2)

    pltpu.sync_copy(m1, sh1.at[sid])
    plsc.subcore_barrier()
    for t in range(_NT):
        pltpu.sync_copy(sh1.at[t], tmp_c)
        if t == 0:
            def cp0(g, _):
                m1[pl.ds(g * 16, 16)] = tmp_c[pl.ds(g * 16, 16)]
                return 0
            lax.fori_loop(0, _NG, cp0, 0, unroll=4)
        else:
            def acc1(g, _):
                sl = pl.ds(g * 16, 16)
                m1[sl] = m1[sl] + tmp_c[sl]
                return 0
            lax.fori_loop(0, _NG, acc1, 0, unroll=4)

    T1 = jnp.int32(_NPIX - _NMIN)

    def sc1(g, carry):
        run, cnt = carry
        v = m1[pl.ds(g * 16, 16)]
        cs = plsc.cumsum(v)
        pe = cs + run - v
        pc = plsc.all_reduce_population_count(pe <= T1)
        return run + jnp.max(cs), cnt + jnp.max(pc)
    _, cnt1 = lax.fori_loop(0, _NG, sc1, (jnp.int32(0), jnp.int32(0)))
    b1s = cnt1 - 1
    b1v = jnp.broadcast_to(b1s, (16,))

    def ca(g, acc):
        v = m1[pl.ds(g * 16, 16)]
        binid = g * 16 + lane
        return acc + jnp.sum(jnp.where(binid > b1v, v, zeros_i))
    c_above = lax.fori_loop(0, _NG, ca, jnp.int32(0))

    _zero_hist(hist_c, zeros_i)

    def zs(g, _):
        hist_s[pl.ds(g * 16, 16)] = zeros_f
        return 0
    lax.fori_loop(0, _NB + 1, zs, 0, unroll=8)

    def chunk2(c, sacc):
        pltpu.sync_copy(loss_hbm.at[pl.ds(base + c * _CH, _CH)], buf)

        def inner(v, sacc):
            for u in range(8):
                x = buf[pl.ds((v * 8 + u) * 16, 16)]
                k = lax.bitcast_convert_type(x, jnp.int32) & jnp.int32(0x7FFFFFFF)
                b1 = lax.shift_right_logical(k, 20)
                eq = b1 == b1v
                b2 = lax.shift_right_logical(k, 9) & jnp.int32(0x7FF)
                idx = b2 * 16 + lane
                plsc.addupdate_scatter(hist_c, [idx], ones_i, mask=eq)
                plsc.addupdate_scatter(hist_s, [idx], x, mask=eq)
                sacc = sacc + jnp.where(b1 > b1v, x, zeros_f)
            return sacc
        return lax.fori_loop(0, _CH // 128, inner, sacc)
    sa_vec = lax.fori_loop(0, _NCH, chunk2, zeros_f)
    hist_s[pl.ds(_NB * 16, 16)] = sa_vec

    def lm2(g, _):
        accc = zeros_i
        accs = zeros_f
        for c in range(16):
            gi = g * 256 + lane * 16 + c
            accc = accc + plsc.load_gather(hist_c, [gi])
            accs = accs + plsc.load_gather(hist_s, [gi])
        m2c[pl.ds(g * 16, 16)] = accc
        m2s[pl.ds(g * 16, 16)] = accs
        return 0
    lax.fori_loop(0, _NG, lm2, 0, unroll=2)
    m2s[pl.ds(_NB, 16)] = hist_s[pl.ds(_NB * 16, 16)]

    pltpu.sync_copy(m2c, sh2c.at[sid])
    pltpu.sync_copy(m2s, sh2s.at[sid])
    plsc.subcore_barrier()
    for t in range(_NT):
        pltpu.sync_copy(sh2c.at[t], tmp_c)
        pltpu.sync_copy(sh2s.at[t], tmp_s)
        if t == 0:
            def cp2(g, _):
                sl = pl.ds(g * 16, 16)
                m2c[sl] = tmp_c[sl]
                return 0
            lax.fori_loop(0, _NG, cp2, 0, unroll=4)

            def cp2s(g, _):
                sl = pl.ds(g * 16, 16)
                m2s[sl] = tmp_s[sl]
                return 0
            lax.fori_loop(0, _NG + 1, cp2s, 0, unroll=4)
        else:
            def acc2(g, _):
                sl = pl.ds(g * 16, 16)
                m2c[sl] = m2c[sl] + tmp_c[sl]
                return 0
            lax.fori_loop(0, _NG, acc2, 0, unroll=4)

            def acc2s(g, _):
                sl = pl.ds(g * 16, 16)
                m2s[sl] = m2s[sl] + tmp_s[sl]
                return 0
            lax.fori_loop(0, _NG + 1, acc2s, 0, unroll=4)
    s_above = jnp.sum(m2s[pl.ds(_NB, 16)])

    total2 = jnp.max(plsc.load_gather(m1, [b1v]))
    n2 = jnp.int32(_NMIN) - c_above
    T2 = total2 - n2

    def sc2(g, carry):
        run, cnt = carry
        v = m2c[pl.ds(g * 16, 16)]
        cs = plsc.cumsum(v)
        pe = cs + run - v
        pc = plsc.all_reduce_population_count(pe <= T2)
        return run + jnp.max(cs), cnt + jnp.max(pc)
    _, cnt2 = lax.fori_loop(0, _NG, sc2, (jnp.int32(0), jnp.int32(0)))
    b2s = cnt2 - 1
    b2v = jnp.broadcast_to(b2s, (16,))

    def suf2(g, carry):
        accc, accs = carry
        binid = g * 16 + lane
        gt = binid > b2v
        accc = accc + jnp.sum(jnp.where(gt, m2c[pl.ds(g * 16, 16)], zeros_i))
        accs = accs + jnp.sum(jnp.where(gt, m2s[pl.ds(g * 16, 16)], zeros_f))
        return accc, accs
    c_hi2, s_hi2 = lax.fori_loop(0, _NG, suf2, (jnp.int32(0), jnp.float32(0.0)))

    c_hi = c_above + c_hi2
    s_hi = s_above + s_hi2
    tbits = jnp.broadcast_to(
        lax.shift_left(b1s, 20) | lax.shift_left(b2s, 9) | jnp.int32(256), (16,))
    t_rep = lax.bitcast_convert_type(tbits, jnp.float32)
    rem = (jnp.int32(_NMIN) - c_hi).astype(jnp.float32)
    res[...] = (s_hi + rem * t_rep) * jnp.float32(1.0 / _NMIN)

    @pl.when((cid == 0) & (sid == 0))
    def _out():
        pltpu.sync_copy(res, out_hbm)


@jax.jit
def _sc_topk_call(loss):
    mesh = plsc.VectorSubcoreMesh(core_axis_name="c", subcore_axis_name="s")
    f = functools.partial(
        pl.kernel,
        out_type=jax.ShapeDtypeStruct((16,), jnp.float32),
        mesh=mesh,
        compiler_params=pltpu.CompilerParams(needs_layout_passes=False),
        scratch_types=[
            pltpu.VMEM((_CH,), jnp.float32),
            pltpu.VMEM((_NB * 16 + 16,), jnp.int32),
            pltpu.VMEM((_NB * 16 + 16,), jnp.float32),
            pltpu.VMEM((_NB,), jnp.int32),
            pltpu.VMEM((_NB,), jnp.int32),
            pltpu.VMEM((_NB + 16,), jnp.float32),
            pltpu.VMEM((_NB,), jnp.int32),
            pltpu.VMEM((_NB + 16,), jnp.float32),
            pltpu.VMEM((16,), jnp.float32),
            pltpu.VMEM_SHARED((_NT, _NB), jnp.int32),
            pltpu.VMEM_SHARED((_NT, _NB), jnp.int32),
            pltpu.VMEM_SHARED((_NT, _NB + 16), jnp.float32),
        ],
    )(_sc_topk_body)
    return f(loss)


def kernel(preds, labels):
    loss, n_hard, hard_sum = _loss_call(preds, labels)
    mean_topk = _sc_topk_call(loss)[0]
    mean_hard = hard_sum / n_hard
    return jnp.where(n_hard < jnp.float32(_NMIN), mean_topk, mean_hard)


# speedup vs baseline: 8.1956x; 1.3516x over previous
import functools

import jax
import jax.numpy as jnp
import numpy as np
from jax import lax
from jax.experimental import pallas as pl
from jax.experimental.pallas import tpu as pltpu
from jax.experimental.pallas import tpu_sc as plsc

_IGNORE = 255
_THRESH = float(-np.log(0.7))

_B, _C, _H, _W = 8, 19, 512, 512
_NPIX = _B * _H * _W
_NMIN = _NPIX // 16
_L = 4096


_BH = 16


def _loss_body(preds_ref, labels_ref, loss_ref, cnt_ref, sum_ref, acc_ref):
    i = pl.program_id(0)
    j = pl.program_id(1)

    @pl.when((i == 0) & (j == 0))
    def _init():
        acc_ref[0] = 0.0
        acc_ref[1] = 0.0

    x = preds_ref[0]
    lab = labels_ref[0]
    m = jnp.max(x, axis=0)
    sh = x - m[None]
    s = jnp.sum(jnp.exp(sh), axis=0)
    logs = jnp.log(s)
    ch = jax.lax.broadcasted_iota(jnp.int32, (_C, _BH, _W), 0)
    picked = jnp.sum(jnp.where(ch == lab[None], sh, 0.0), axis=0)
    nll = logs - picked
    valid = lab != _IGNORE
    loss = jnp.where(valid, nll, 0.0)
    loss_ref[0] = loss

    hard = loss > _THRESH
    acc_ref[0] += jnp.sum(jnp.where(hard, 1.0, 0.0))
    acc_ref[1] += jnp.sum(jnp.where(hard, loss, 0.0))

    @pl.when((i == _B - 1) & (j == _H // _BH - 1))
    def _fin():
        cnt_ref[0, 0] = acc_ref[0]
        sum_ref[0, 0] = acc_ref[1]


@functools.partial(jax.jit, static_argnames=("interpret",))
def _loss_call(preds, labels, interpret=False):
    grid = (_B, _H // _BH)
    loss, cnt, hsum = pl.pallas_call(
        _loss_body,
        grid=grid,
        in_specs=[
            pl.BlockSpec((1, _C, _BH, _W), lambda i, j: (i, 0, j, 0)),
            pl.BlockSpec((1, _BH, _W), lambda i, j: (i, j, 0)),
        ],
        out_specs=[
            pl.BlockSpec((1, _BH, _W), lambda i, j: (i, j, 0)),
            pl.BlockSpec(memory_space=pltpu.SMEM),
            pl.BlockSpec(memory_space=pltpu.SMEM),
        ],
        out_shape=[
            jax.ShapeDtypeStruct((_B, _H, _W), jnp.float32),
            jax.ShapeDtypeStruct((1, 1), jnp.float32),
            jax.ShapeDtypeStruct((1, 1), jnp.float32),
        ],
        scratch_shapes=[pltpu.SMEM((2,), jnp.float32)],
        interpret=interpret,
    )(preds, labels)
    return loss.reshape(-1), cnt[0, 0], hsum[0, 0]


_NT = 16
_PT = _NPIX // _NT
_CH = 8192
_NCH = _PT // _CH
_NB = 2048
_NG = _NB // 16


def _sc_topk_body(loss_hbm, out_hbm, buf, hist_c, m1, m2c, tmp_c, rsum,
                  sabuf, sibuf, res, sh1, sh2c, sh_sa, sh_si):
    cid = lax.axis_index("c")
    sid = lax.axis_index("s")
    lane = lax.iota(jnp.int32, 16)
    base = sid * _PT
    ones_i = jnp.ones((16,), jnp.int32)
    zeros_i = jnp.zeros((16,), jnp.int32)
    zeros_f = jnp.zeros((16,), jnp.float32)

    def _zero_hist(ref):
        @plsc.parallel_loop(0, _NB, unroll=8)
        def zb(g):
            ref[pl.ds(g * 16, 16)] = zeros_i

    _zero_hist(hist_c)

    def chunk1(c, _):
        pltpu.sync_copy(loss_hbm.at[pl.ds(base + c * _CH, _CH)], buf)

        @plsc.parallel_loop(0, _CH // 16, unroll=8)
        def inner(v):
            x = buf[pl.ds(v * 16, 16)]
            k = lax.bitcast_convert_type(x, jnp.int32) & jnp.int32(0x7FFFFFFF)
            b1 = lax.shift_right_logical(k, 20)
            plsc.addupdate_scatter(hist_c, [b1 * 16 + lane], ones_i)
        return 0
    lax.fori_loop(0, _NCH, chunk1, 0)

    @plsc.parallel_loop(0, _NG, unroll=2)
    def lm1(g):
        acc = zeros_i
        for c in range(16):
            acc = acc + plsc.load_gather(hist_c, [g * 256 + lane * 16 + c])
        m1[pl.ds(g * 16, 16)] = acc

    pltpu.sync_copy(m1, sh1.at[sid])
    plsc.subcore_barrier()
    for t in range(_NT):
        pltpu.sync_copy(sh1.at[t], tmp_c)
        if t == 0:
            @plsc.parallel_loop(0, _NG, unroll=4)
            def cp0(g):
                m1[pl.ds(g * 16, 16)] = tmp_c[pl.ds(g * 16, 16)]
        else:
            @plsc.parallel_loop(0, _NG, unroll=4)
            def acc1(g):
                sl = pl.ds(g * 16, 16)
                m1[sl] = m1[sl] + tmp_c[sl]

    T1 = jnp.int32(_NPIX - _NMIN)

    def sc1(g, carry):
        run, cnt = carry
        v = m1[pl.ds(g * 16, 16)]
        cs = plsc.cumsum(v)
        pe = cs + run - v
        pc = plsc.all_reduce_population_count(pe <= T1)
        return run + jnp.max(cs), cnt + jnp.max(pc)
    _, cnt1 = lax.fori_loop(0, _NG, sc1, (jnp.int32(0), jnp.int32(0)))
    b1s = cnt1 - 1
    b1v = jnp.broadcast_to(b1s, (16,))

    def ca(g, acc):
        v = m1[pl.ds(g * 16, 16)]
        binid = g * 16 + lane
        return acc + jnp.sum(jnp.where(binid > b1v, v, zeros_i))
    c_above = lax.fori_loop(0, _NG, ca, jnp.int32(0))

    _zero_hist(hist_c)

    def chunk2(c, sacc):
        pltpu.sync_copy(loss_hbm.at[pl.ds(base + c * _CH, _CH)], buf)

        def inner(v, sacc):
            x = buf[pl.ds(v * 16, 16)]
            k = lax.bitcast_convert_type(x, jnp.int32) & jnp.int32(0x7FFFFFFF)
            b1 = lax.shift_right_logical(k, 20)
            eq = b1 == b1v
            b2 = lax.shift_right_logical(k, 9) & jnp.int32(0x7FF)
            plsc.addupdate_scatter(hist_c, [b2 * 16 + lane], ones_i, mask=eq)
            return sacc + jnp.where(b1 > b1v, x, zeros_f)
        return plsc.parallel_loop(0, _CH // 16, unroll=8, carry=sacc)(inner)
    sa_vec = lax.fori_loop(0, _NCH, chunk2, zeros_f)

    @plsc.parallel_loop(0, _NG, unroll=2)
    def lm2(g):
        acc = zeros_i
        for c in range(16):
            acc = acc + plsc.load_gather(hist_c, [g * 256 + lane * 16 + c])
        m2c[pl.ds(g * 16, 16)] = acc

    pltpu.sync_copy(m2c, sh2c.at[sid])
    plsc.subcore_barrier()
    for t in range(_NT):
        pltpu.sync_copy(sh2c.at[t], tmp_c)
        if t == 0:
            @plsc.parallel_loop(0, _NG, unroll=4)
            def cp2(g):
                m2c[pl.ds(g * 16, 16)] = tmp_c[pl.ds(g * 16, 16)]
        else:
            @plsc.parallel_loop(0, _NG, unroll=4)
            def acc2(g):
                sl = pl.ds(g * 16, 16)
                m2c[sl] = m2c[sl] + tmp_c[sl]

    total2 = jnp.max(plsc.load_gather(m1, [b1v]))
    n2 = jnp.int32(_NMIN) - c_above
    T2 = total2 - n2

    def sc2(g, carry):
        run, cnt = carry
        v = m2c[pl.ds(g * 16, 16)]
        cs = plsc.cumsum(v)
        pe = cs + run - v
        pc = plsc.all_reduce_population_count(pe <= T2)
        return run + jnp.max(cs), cnt + jnp.max(pc)
    _, cnt2 = lax.fori_loop(0, _NG, sc2, (jnp.int32(0), jnp.int32(0)))
    b2s = cnt2 - 1
    b2v = jnp.broadcast_to(b2s, (16,))

    def suf2(g, acc):
        binid = g * 16 + lane
        gt = binid > b2v
        return acc + jnp.sum(jnp.where(gt, m2c[pl.ds(g * 16, 16)], zeros_i))
    c_hi2 = lax.fori_loop(0, _NG, suf2, jnp.int32(0))

    def chunk3(c, sacc):
        pltpu.sync_copy(loss_hbm.at[pl.ds(base + c * _CH, _CH)], buf)

        def inner(v, sacc):
            x = buf[pl.ds(v * 16, 16)]
            k = lax.bitcast_convert_type(x, jnp.int32) & jnp.int32(0x7FFFFFFF)
            b1 = lax.shift_right_logical(k, 20)
            b2 = lax.shift_right_logical(k, 9) & jnp.int32(0x7FF)
            m = (b1 == b1v) & (b2 > b2v)
            return sacc + jnp.where(m, x, zeros_f)
        return plsc.parallel_loop(0, _CH // 16, unroll=8, carry=sacc)(inner)
    si_vec = lax.fori_loop(0, _NCH, chunk3, zeros_f)

    sabuf[...] = sa_vec
    sibuf[...] = si_vec
    pltpu.sync_copy(sabuf, sh_sa.at[pl.ds(sid * 16, 16)])
    pltpu.sync_copy(sibuf, sh_si.at[pl.ds(sid * 16, 16)])
    plsc.subcore_barrier()
    pltpu.sync_copy(sh_sa, rsum)
    s_above = zeros_f
    for t in range(_NT):
        s_above = s_above + rsum[pl.ds(t * 16, 16)]
    pltpu.sync_copy(sh_si, rsum)
    s_in = zeros_f
    for t in range(_NT):
        s_in = s_in + rsum[pl.ds(t * 16, 16)]
    s_hi = jnp.sum(s_above) + jnp.sum(s_in)

    c_hi = c_above + c_hi2
    tbits = jnp.broadcast_to(
        lax.shift_left(b1s, 20) | lax.shift_left(b2s, 9) | jnp.int32(256), (16,))
    t_rep = lax.bitcast_convert_type(tbits, jnp.float32)
    rem = (jnp.int32(_NMIN) - c_hi).astype(jnp.float32)
    res[...] = (s_hi + rem * t_rep) * jnp.float32(1.0 / _NMIN)

    @pl.when((cid == 0) & (sid == 0))
    def _out():
        pltpu.sync_copy(res, out_hbm)


@jax.jit
def _sc_topk_call(loss):
    mesh = plsc.VectorSubcoreMesh(core_axis_name="c", subcore_axis_name="s")
    f = functools.partial(
        pl.kernel,
        out_type=jax.ShapeDtypeStruct((16,), jnp.float32),
        mesh=mesh,
        compiler_params=pltpu.CompilerParams(needs_layout_passes=False),
        scratch_types=[
            pltpu.VMEM((_CH,), jnp.float32),
            pltpu.VMEM((_NB * 16,), jnp.int32),
            pltpu.VMEM((_NB,), jnp.int32),
            pltpu.VMEM((_NB,), jnp.int32),
            pltpu.VMEM((_NB,), jnp.int32),
            pltpu.VMEM((_NT * 16,), jnp.float32),
            pltpu.VMEM((16,), jnp.float32),
            pltpu.VMEM((16,), jnp.float32),
            pltpu.VMEM((16,), jnp.float32),
            pltpu.VMEM_SHARED((_NT, _NB), jnp.int32),
            pltpu.VMEM_SHARED((_NT, _NB), jnp.int32),
            pltpu.VMEM_SHARED((_NT * 16,), jnp.float32),
            pltpu.VMEM_SHARED((_NT * 16,), jnp.float32),
        ],
    )(_sc_topk_body)
    return f(loss)


def kernel(preds, labels):
    loss, n_hard, hard_sum = _loss_call(preds, labels)
    mean_topk = _sc_topk_call(loss)[0]
    mean_hard = hard_sum / n_hard
    return jnp.where(n_hard < jnp.float32(_NMIN), mean_topk, mean_hard)


# speedup vs baseline: 10.2534x; 1.2511x over previous
import functools

import jax
import jax.numpy as jnp
import numpy as np
from jax import lax
from jax.experimental import pallas as pl
from jax.experimental.pallas import tpu as pltpu
from jax.experimental.pallas import tpu_sc as plsc

_IGNORE = 255
_THRESH = float(-np.log(0.7))

_B, _C, _H, _W = 8, 19, 512, 512
_NPIX = _B * _H * _W
_NMIN = _NPIX // 16
_L = 4096


_BH = 32


def _loss_body(preds_ref, labels_ref, loss_ref, cnt_ref, sum_ref, acc_ref):
    i = pl.program_id(0)
    j = pl.program_id(1)

    @pl.when((i == 0) & (j == 0))
    def _init():
        acc_ref[0] = 0.0
        acc_ref[1] = 0.0

    x = preds_ref[0]
    lab = labels_ref[0]
    m = jnp.max(x, axis=0)
    sh = x - m[None]
    s = jnp.sum(jnp.exp(sh), axis=0)
    logs = jnp.log(s)
    ch = jax.lax.broadcasted_iota(jnp.int32, (_C, _BH, _W), 0)
    picked = jnp.sum(jnp.where(ch == lab[None], sh, 0.0), axis=0)
    nll = logs - picked
    valid = lab != _IGNORE
    loss = jnp.where(valid, nll, 0.0)
    loss_ref[0] = loss

    hard = loss > _THRESH
    acc_ref[0] += jnp.sum(jnp.where(hard, 1.0, 0.0))
    acc_ref[1] += jnp.sum(jnp.where(hard, loss, 0.0))

    @pl.when((i == _B - 1) & (j == _H // _BH - 1))
    def _fin():
        cnt_ref[0, 0] = acc_ref[0]
        sum_ref[0, 0] = acc_ref[1]


@functools.partial(jax.jit, static_argnames=("interpret",))
def _loss_call(preds, labels, interpret=False):
    grid = (_B, _H // _BH)
    loss, cnt, hsum = pl.pallas_call(
        _loss_body,
        grid=grid,
        in_specs=[
            pl.BlockSpec((1, _C, _BH, _W), lambda i, j: (i, 0, j, 0)),
            pl.BlockSpec((1, _BH, _W), lambda i, j: (i, j, 0)),
        ],
        out_specs=[
            pl.BlockSpec((1, _BH, _W), lambda i, j: (i, j, 0)),
            pl.BlockSpec(memory_space=pltpu.SMEM),
            pl.BlockSpec(memory_space=pltpu.SMEM),
        ],
        out_shape=[
            jax.ShapeDtypeStruct((_B, _H, _W), jnp.float32),
            jax.ShapeDtypeStruct((1, 1), jnp.float32),
            jax.ShapeDtypeStruct((1, 1), jnp.float32),
        ],
        scratch_shapes=[pltpu.SMEM((2,), jnp.float32)],
        interpret=interpret,
    )(preds, labels)
    return loss.reshape(-1), cnt[0, 0], hsum[0, 0]


_NT = 16
_PT = _NPIX // _NT
_CH = 8192
_NCH = _PT // _CH
_NB = 2048
_NG = _NB // 16


def _sc_topk_body(loss_hbm, out_hbm, buf, hist_c, m1, m2c, tmp_c, rsum,
                  sabuf, sibuf, res, sh1, sh2c, sh_sa, sh_si):
    cid = lax.axis_index("c")
    sid = lax.axis_index("s")
    lane = lax.iota(jnp.int32, 16)
    base = sid * _PT
    ones_i = jnp.ones((16,), jnp.int32)
    zeros_i = jnp.zeros((16,), jnp.int32)
    zeros_f = jnp.zeros((16,), jnp.float32)

    def _zero_hist(ref):
        @plsc.parallel_loop(0, _NB, unroll=8)
        def zb(g):
            ref[pl.ds(g * 16, 16)] = zeros_i

    _zero_hist(hist_c)

    def chunk1(c, _):
        pltpu.sync_copy(loss_hbm.at[pl.ds(base + c * _CH, _CH)], buf)

        @plsc.parallel_loop(0, _CH // 16, unroll=8)
        def inner(v):
            x = buf[pl.ds(v * 16, 16)]
            k = lax.bitcast_convert_type(x, jnp.int32) & jnp.int32(0x7FFFFFFF)
            b1 = lax.shift_right_logical(k, 20)
            plsc.addupdate_scatter(hist_c, [b1 * 16 + lane], ones_i)
        return 0
    lax.fori_loop(0, _NCH, chunk1, 0)

    @plsc.parallel_loop(0, _NG, unroll=2)
    def lm1(g):
        acc = zeros_i
        for c in range(16):
            acc = acc + plsc.load_gather(hist_c, [g * 256 + lane * 16 + c])
        m1[pl.ds(g * 16, 16)] = acc

    pltpu.sync_copy(m1, sh1.at[sid])
    plsc.subcore_barrier()
    for t in range(_NT):
        pltpu.sync_copy(sh1.at[t], tmp_c)
        if t == 0:
            @plsc.parallel_loop(0, _NG, unroll=4)
            def cp0(g):
                m1[pl.ds(g * 16, 16)] = tmp_c[pl.ds(g * 16, 16)]
        else:
            @plsc.parallel_loop(0, _NG, unroll=4)
            def acc1(g):
                sl = pl.ds(g * 16, 16)
                m1[sl] = m1[sl] + tmp_c[sl]

    T1 = jnp.int32(_NPIX - _NMIN)

    def sc1(g, carry):
        run, cnt = carry
        v = m1[pl.ds(g * 16, 16)]
        cs = plsc.cumsum(v)
        pe = cs + run - v
        pc = plsc.all_reduce_population_count(pe <= T1)
        return run + jnp.max(cs), cnt + jnp.max(pc)
    _, cnt1 = lax.fori_loop(0, _NG, sc1, (jnp.int32(0), jnp.int32(0)))
    b1s = cnt1 - 1
    b1v = jnp.broadcast_to(b1s, (16,))

    def ca(g, acc):
        v = m1[pl.ds(g * 16, 16)]
        binid = g * 16 + lane
        return acc + jnp.sum(jnp.where(binid > b1v, v, zeros_i))
    c_above = lax.fori_loop(0, _NG, ca, jnp.int32(0))

    _zero_hist(hist_c)

    def chunk2(c, sacc):
        pltpu.sync_copy(loss_hbm.at[pl.ds(base + c * _CH, _CH)], buf)

        def inner(v, sacc):
            x = buf[pl.ds(v * 16, 16)]
            k = lax.bitcast_convert_type(x, jnp.int32) & jnp.int32(0x7FFFFFFF)
            b1 = lax.shift_right_logical(k, 20)
            eq = b1 == b1v
            b2 = lax.shift_right_logical(k, 9) & jnp.int32(0x7FF)
            plsc.addupdate_scatter(hist_c, [b2 * 16 + lane], ones_i, mask=eq)
            return sacc + jnp.where(b1 > b1v, x, zeros_f)
        return plsc.parallel_loop(0, _CH // 16, unroll=8, carry=sacc)(inner)
    sa_vec = lax.fori_loop(0, _NCH, chunk2, zeros_f)

    @plsc.parallel_loop(0, _NG, unroll=2)
    def lm2(g):
        acc = zeros_i
        for c in range(16):
            acc = acc + plsc.load_gather(hist_c, [g * 256 + lane * 16 + c])
        m2c[pl.ds(g * 16, 16)] = acc

    pltpu.sync_copy(m2c, sh2c.at[sid])
    plsc.subcore_barrier()
    for t in range(_NT):
        pltpu.sync_copy(sh2c.at[t], tmp_c)
        if t == 0:
            @plsc.parallel_loop(0, _NG, unroll=4)
            def cp2(g):
                m2c[pl.ds(g * 16, 16)] = tmp_c[pl.ds(g * 16, 16)]
        else:
            @plsc.parallel_loop(0, _NG, unroll=4)
            def acc2(g):
                sl = pl.ds(g * 16, 16)
                m2c[sl] = m2c[sl] + tmp_c[sl]

    total2 = jnp.max(plsc.load_gather(m1, [b1v]))
    n2 = jnp.int32(_NMIN) - c_above
    T2 = total2 - n2

    def sc2(g, carry):
        run, cnt = carry
        v = m2c[pl.ds(g * 16, 16)]
        cs = plsc.cumsum(v)
        pe = cs + run - v
        pc = plsc.all_reduce_population_count(pe <= T2)
        return run + jnp.max(cs), cnt + jnp.max(pc)
    _, cnt2 = lax.fori_loop(0, _NG, sc2, (jnp.int32(0), jnp.int32(0)))
    b2s = cnt2 - 1
    b2v = jnp.broadcast_to(b2s, (16,))

    def suf2(g, acc):
        binid = g * 16 + lane
        gt = binid > b2v
        return acc + jnp.sum(jnp.where(gt, m2c[pl.ds(g * 16, 16)], zeros_i))
    c_hi2 = lax.fori_loop(0, _NG, suf2, jnp.int32(0))

    def chunk3(c, sacc):
        pltpu.sync_copy(loss_hbm.at[pl.ds(base + c * _CH, _CH)], buf)

        def inner(v, sacc):
            x = buf[pl.ds(v * 16, 16)]
            k = lax.bitcast_convert_type(x, jnp.int32) & jnp.int32(0x7FFFFFFF)
            b1 = lax.shift_right_logical(k, 20)
            b2 = lax.shift_right_logical(k, 9) & jnp.int32(0x7FF)
            m = (b1 == b1v) & (b2 > b2v)
            return sacc + jnp.where(m, x, zeros_f)
        return plsc.parallel_loop(0, _CH // 16, unroll=8, carry=sacc)(inner)
    si_vec = lax.fori_loop(0, _NCH, chunk3, zeros_f)

    sabuf[...] = sa_vec
    sibuf[...] = si_vec
    pltpu.sync_copy(sabuf, sh_sa.at[pl.ds(sid * 16, 16)])
    pltpu.sync_copy(sibuf, sh_si.at[pl.ds(sid * 16, 16)])
    plsc.subcore_barrier()
    pltpu.sync_copy(sh_sa, rsum)
    s_above = zeros_f
    for t in range(_NT):
        s_above = s_above + rsum[pl.ds(t * 16, 16)]
    pltpu.sync_copy(sh_si, rsum)
    s_in = zeros_f
    for t in range(_NT):
        s_in = s_in + rsum[pl.ds(t * 16, 16)]
    s_hi = jnp.sum(s_above) + jnp.sum(s_in)

    c_hi = c_above + c_hi2
    tbits = jnp.broadcast_to(
        lax.shift_left(b1s, 20) | lax.shift_left(b2s, 9) | jnp.int32(256), (16,))
    t_rep = lax.bitcast_convert_type(tbits, jnp.float32)
    rem = (jnp.int32(_NMIN) - c_hi).astype(jnp.float32)
    res[...] = (s_hi + rem * t_rep) * jnp.float32(1.0 / _NMIN)

    @pl.when((cid == 0) & (sid == 0))
    def _out():
        pltpu.sync_copy(res, out_hbm)


@jax.jit
def _sc_topk_call(loss):
    mesh = plsc.VectorSubcoreMesh(core_axis_name="c", subcore_axis_name="s")
    f = functools.partial(
        pl.kernel,
        out_type=jax.ShapeDtypeStruct((16,), jnp.float32),
        mesh=mesh,
        compiler_params=pltpu.CompilerParams(needs_layout_passes=False),
        scratch_types=[
            pltpu.VMEM((_CH,), jnp.float32),
            pltpu.VMEM((_NB * 16,), jnp.int32),
            pltpu.VMEM((_NB,), jnp.int32),
            pltpu.VMEM((_NB,), jnp.int32),
            pltpu.VMEM((_NB,), jnp.int32),
            pltpu.VMEM((_NT * 16,), jnp.float32),
            pltpu.VMEM((16,), jnp.float32),
            pltpu.VMEM((16,), jnp.float32),
            pltpu.VMEM((16,), jnp.float32),
            pltpu.VMEM_SHARED((_NT, _NB), jnp.int32),
            pltpu.VMEM_SHARED((_NT, _NB), jnp.int32),
            pltpu.VMEM_SHARED((_NT * 16,), jnp.float32),
            pltpu.VMEM_SHARED((_NT * 16,), jnp.float32),
        ],
    )(_sc_topk_body)
    return f(loss)


def kernel(preds, labels):
    loss, n_hard, hard_sum = _loss_call(preds, labels)
    mean_topk = _sc_topk_call(loss)[0]
    mean_hard = hard_sum / n_hard
    return jnp.where(n_hard < jnp.float32(_NMIN), mean_topk, mean_hard)


# speedup vs baseline: 11.7237x; 1.1434x over previous
import functools

import jax
import jax.numpy as jnp
import numpy as np
from jax import lax
from jax.experimental import pallas as pl
from jax.experimental.pallas import tpu as pltpu
from jax.experimental.pallas import tpu_sc as plsc

_IGNORE = 255
_THRESH = float(-np.log(0.7))

_B, _C, _H, _W = 8, 19, 512, 512
_NPIX = _B * _H * _W
_NMIN = _NPIX // 16
_L = 4096


_BH = 64


def _loss_body(preds_ref, labels_ref, loss_ref, cnt_ref, sum_ref, acc_ref):
    i = pl.program_id(0)
    j = pl.program_id(1)

    @pl.when((i == 0) & (j == 0))
    def _init():
        acc_ref[0] = 0.0
        acc_ref[1] = 0.0

    x = preds_ref[0]
    lab = labels_ref[0]
    m = jnp.max(x, axis=0)
    sh = x - m[None]
    s = jnp.sum(jnp.exp(sh), axis=0)
    logs = jnp.log(s)
    ch = jax.lax.broadcasted_iota(jnp.int32, (_C, _BH, _W), 0)
    picked = jnp.sum(jnp.where(ch == lab[None], sh, 0.0), axis=0)
    nll = logs - picked
    valid = lab != _IGNORE
    loss = jnp.where(valid, nll, 0.0)
    loss_ref[0] = loss

    hard = loss > _THRESH
    acc_ref[0] += jnp.sum(jnp.where(hard, 1.0, 0.0))
    acc_ref[1] += jnp.sum(jnp.where(hard, loss, 0.0))

    @pl.when((i == _B - 1) & (j == _H // _BH - 1))
    def _fin():
        cnt_ref[0, 0] = acc_ref[0]
        sum_ref[0, 0] = acc_ref[1]


@functools.partial(jax.jit, static_argnames=("interpret",))
def _loss_call(preds, labels, interpret=False):
    grid = (_B, _H // _BH)
    loss, cnt, hsum = pl.pallas_call(
        _loss_body,
        grid=grid,
        in_specs=[
            pl.BlockSpec((1, _C, _BH, _W), lambda i, j: (i, 0, j, 0)),
            pl.BlockSpec((1, _BH, _W), lambda i, j: (i, j, 0)),
        ],
        out_specs=[
            pl.BlockSpec((1, _BH, _W), lambda i, j: (i, j, 0)),
            pl.BlockSpec(memory_space=pltpu.SMEM),
            pl.BlockSpec(memory_space=pltpu.SMEM),
        ],
        out_shape=[
            jax.ShapeDtypeStruct((_B, _H, _W), jnp.float32),
            jax.ShapeDtypeStruct((1, 1), jnp.float32),
            jax.ShapeDtypeStruct((1, 1), jnp.float32),
        ],
        scratch_shapes=[pltpu.SMEM((2,), jnp.float32)],
        interpret=interpret,
    )(preds, labels)
    return loss.reshape(-1), cnt[0, 0], hsum[0, 0]


_NT = 16
_PT = _NPIX // _NT
_CH = 8192
_NCH = _PT // _CH
_NB = 2048
_NG = _NB // 16


def _sc_topk_body(loss_hbm, out_hbm, buf, hist_c, m1, m2c, tmp_c, rsum,
                  sabuf, sibuf, res, sh1, sh2c, sh_sa, sh_si):
    cid = lax.axis_index("c")
    sid = lax.axis_index("s")
    lane = lax.iota(jnp.int32, 16)
    base = sid * _PT
    ones_i = jnp.ones((16,), jnp.int32)
    zeros_i = jnp.zeros((16,), jnp.int32)
    zeros_f = jnp.zeros((16,), jnp.float32)

    def _zero_hist(ref):
        @plsc.parallel_loop(0, _NB, unroll=8)
        def zb(g):
            ref[pl.ds(g * 16, 16)] = zeros_i

    _zero_hist(hist_c)

    def chunk1(c, _):
        pltpu.sync_copy(loss_hbm.at[pl.ds(base + c * _CH, _CH)], buf)

        @plsc.parallel_loop(0, _CH // 16, unroll=8)
        def inner(v):
            x = buf[pl.ds(v * 16, 16)]
            k = lax.bitcast_convert_type(x, jnp.int32) & jnp.int32(0x7FFFFFFF)
            b1 = lax.shift_right_logical(k, 20)
            plsc.addupdate_scatter(hist_c, [b1 * 16 + lane], ones_i)
        return 0
    lax.fori_loop(0, _NCH, chunk1, 0)

    @plsc.parallel_loop(0, _NG, unroll=2)
    def lm1(g):
        acc = zeros_i
        for c in range(16):
            acc = acc + plsc.load_gather(hist_c, [g * 256 + lane * 16 + c])
        m1[pl.ds(g * 16, 16)] = acc

    pltpu.sync_copy(m1, sh1.at[sid])
    plsc.subcore_barrier()
    for t in range(_NT):
        pltpu.sync_copy(sh1.at[t], tmp_c)
        if t == 0:
            @plsc.parallel_loop(0, _NG, unroll=4)
            def cp0(g):
                m1[pl.ds(g * 16, 16)] = tmp_c[pl.ds(g * 16, 16)]
        else:
            @plsc.parallel_loop(0, _NG, unroll=4)
            def acc1(g):
                sl = pl.ds(g * 16, 16)
                m1[sl] = m1[sl] + tmp_c[sl]

    T1 = jnp.int32(_NPIX - _NMIN)

    def sc1(g, carry):
        run, cnt = carry
        v = m1[pl.ds(g * 16, 16)]
        cs = plsc.cumsum(v)
        pe = cs + run - v
        pc = plsc.all_reduce_population_count(pe <= T1)
        return run + jnp.max(cs), cnt + jnp.max(pc)
    _, cnt1 = lax.fori_loop(0, _NG, sc1, (jnp.int32(0), jnp.int32(0)))
    b1s = cnt1 - 1
    b1v = jnp.broadcast_to(b1s, (16,))

    def ca(g, acc):
        v = m1[pl.ds(g * 16, 16)]
        binid = g * 16 + lane
        return acc + jnp.sum(jnp.where(binid > b1v, v, zeros_i))
    c_above = lax.fori_loop(0, _NG, ca, jnp.int32(0))

    _zero_hist(hist_c)

    def chunk2(c, sacc):
        pltpu.sync_copy(loss_hbm.at[pl.ds(base + c * _CH, _CH)], buf)

        def inner(v, sacc):
            x = buf[pl.ds(v * 16, 16)]
            k = lax.bitcast_convert_type(x, jnp.int32) & jnp.int32(0x7FFFFFFF)
            b1 = lax.shift_right_logical(k, 20)
            eq = b1 == b1v
            b2 = lax.shift_right_logical(k, 9) & jnp.int32(0x7FF)
            plsc.addupdate_scatter(hist_c, [b2 * 16 + lane], ones_i, mask=eq)
            return sacc + jnp.where(b1 > b1v, x, zeros_f)
        return plsc.parallel_loop(0, _CH // 16, unroll=8, carry=sacc)(inner)
    sa_vec = lax.fori_loop(0, _NCH, chunk2, zeros_f)

    @plsc.parallel_loop(0, _NG, unroll=2)
    def lm2(g):
        acc = zeros_i
        for c in range(16):
            acc = acc + plsc.load_gather(hist_c, [g * 256 + lane * 16 + c])
        m2c[pl.ds(g * 16, 16)] = acc

    pltpu.sync_copy(m2c, sh2c.at[sid])
    plsc.subcore_barrier()
    for t in range(_NT):
        pltpu.sync_copy(sh2c.at[t], tmp_c)
        if t == 0:
            @plsc.parallel_loop(0, _NG, unroll=4)
            def cp2(g):
                m2c[pl.ds(g * 16, 16)] = tmp_c[pl.ds(g * 16, 16)]
        else:
            @plsc.parallel_loop(0, _NG, unroll=4)
            def acc2(g):
                sl = pl.ds(g * 16, 16)
                m2c[sl] = m2c[sl] + tmp_c[sl]

    total2 = jnp.max(plsc.load_gather(m1, [b1v]))
    n2 = jnp.int32(_NMIN) - c_above
    T2 = total2 - n2

    def sc2(g, carry):
        run, cnt = carry
        v = m2c[pl.ds(g * 16, 16)]
        cs = plsc.cumsum(v)
        pe = cs + run - v
        pc = plsc.all_reduce_population_count(pe <= T2)
        return run + jnp.max(cs), cnt + jnp.max(pc)
    _, cnt2 = lax.fori_loop(0, _NG, sc2, (jnp.int32(0), jnp.int32(0)))
    b2s = cnt2 - 1
    b2v = jnp.broadcast_to(b2s, (16,))

    def suf2(g, acc):
        binid = g * 16 + lane
        gt = binid > b2v
        return acc + jnp.sum(jnp.where(gt, m2c[pl.ds(g * 16, 16)], zeros_i))
    c_hi2 = lax.fori_loop(0, _NG, suf2, jnp.int32(0))

    def chunk3(c, sacc):
        pltpu.sync_copy(loss_hbm.at[pl.ds(base + c * _CH, _CH)], buf)

        def inner(v, sacc):
            x = buf[pl.ds(v * 16, 16)]
            k = lax.bitcast_convert_type(x, jnp.int32) & jnp.int32(0x7FFFFFFF)
            b1 = lax.shift_right_logical(k, 20)
            b2 = lax.shift_right_logical(k, 9) & jnp.int32(0x7FF)
            m = (b1 == b1v) & (b2 > b2v)
            return sacc + jnp.where(m, x, zeros_f)
        return plsc.parallel_loop(0, _CH // 16, unroll=8, carry=sacc)(inner)
    si_vec = lax.fori_loop(0, _NCH, chunk3, zeros_f)

    sabuf[...] = sa_vec
    sibuf[...] = si_vec
    pltpu.sync_copy(sabuf, sh_sa.at[pl.ds(sid * 16, 16)])
    pltpu.sync_copy(sibuf, sh_si.at[pl.ds(sid * 16, 16)])
    plsc.subcore_barrier()
    pltpu.sync_copy(sh_sa, rsum)
    s_above = zeros_f
    for t in range(_NT):
        s_above = s_above + rsum[pl.ds(t * 16, 16)]
    pltpu.sync_copy(sh_si, rsum)
    s_in = zeros_f
    for t in range(_NT):
        s_in = s_in + rsum[pl.ds(t * 16, 16)]
    s_hi = jnp.sum(s_above) + jnp.sum(s_in)

    c_hi = c_above + c_hi2
    tbits = jnp.broadcast_to(
        lax.shift_left(b1s, 20) | lax.shift_left(b2s, 9) | jnp.int32(256), (16,))
    t_rep = lax.bitcast_convert_type(tbits, jnp.float32)
    rem = (jnp.int32(_NMIN) - c_hi).astype(jnp.float32)
    res[...] = (s_hi + rem * t_rep) * jnp.float32(1.0 / _NMIN)

    @pl.when((cid == 0) & (sid == 0))
    def _out():
        pltpu.sync_copy(res, out_hbm)


@jax.jit
def _sc_topk_call(loss):
    mesh = plsc.VectorSubcoreMesh(core_axis_name="c", subcore_axis_name="s")
    f = functools.partial(
        pl.kernel,
        out_type=jax.ShapeDtypeStruct((16,), jnp.float32),
        mesh=mesh,
        compiler_params=pltpu.CompilerParams(needs_layout_passes=False),
        scratch_types=[
            pltpu.VMEM((_CH,), jnp.float32),
            pltpu.VMEM((_NB * 16,), jnp.int32),
            pltpu.VMEM((_NB,), jnp.int32),
            pltpu.VMEM((_NB,), jnp.int32),
            pltpu.VMEM((_NB,), jnp.int32),
            pltpu.VMEM((_NT * 16,), jnp.float32),
            pltpu.VMEM((16,), jnp.float32),
            pltpu.VMEM((16,), jnp.float32),
            pltpu.VMEM((16,), jnp.float32),
            pltpu.VMEM_SHARED((_NT, _NB), jnp.int32),
            pltpu.VMEM_SHARED((_NT, _NB), jnp.int32),
            pltpu.VMEM_SHARED((_NT * 16,), jnp.float32),
            pltpu.VMEM_SHARED((_NT * 16,), jnp.float32),
        ],
    )(_sc_topk_body)
    return f(loss)


def kernel(preds, labels):
    loss, n_hard, hard_sum = _loss_call(preds, labels)
    mean_topk = _sc_topk_call(loss)[0]
    mean_hard = hard_sum / n_hard
    return jnp.where(n_hard < jnp.float32(_NMIN), mean_topk, mean_hard)


# speedup vs baseline: 12.5092x; 1.0670x over previous
import functools

import jax
import jax.numpy as jnp
import numpy as np
from jax import lax
from jax.experimental import pallas as pl
from jax.experimental.pallas import tpu as pltpu
from jax.experimental.pallas import tpu_sc as plsc

_IGNORE = 255
_THRESH = float(-np.log(0.7))

_B, _C, _H, _W = 8, 19, 512, 512
_NPIX = _B * _H * _W
_NMIN = _NPIX // 16
_L = 4096


_BH = 128


def _loss_body(preds_ref, labels_ref, loss_ref, cnt_ref, sum_ref, acc_ref):
    i = pl.program_id(0)
    j = pl.program_id(1)

    @pl.when((i == 0) & (j == 0))
    def _init():
        acc_ref[0] = 0.0
        acc_ref[1] = 0.0

    x = preds_ref[0]
    lab = labels_ref[0]
    m = jnp.max(x, axis=0)
    sh = x - m[None]
    s = jnp.sum(jnp.exp(sh), axis=0)
    logs = jnp.log(s)
    ch = jax.lax.broadcasted_iota(jnp.int32, (_C, _BH, _W), 0)
    picked = jnp.sum(jnp.where(ch == lab[None], sh, 0.0), axis=0)
    nll = logs - picked
    valid = lab != _IGNORE
    loss = jnp.where(valid, nll, 0.0)
    loss_ref[0] = loss

    hard = loss > _THRESH
    acc_ref[0] += jnp.sum(jnp.where(hard, 1.0, 0.0))
    acc_ref[1] += jnp.sum(jnp.where(hard, loss, 0.0))

    @pl.when((i == _B - 1) & (j == _H // _BH - 1))
    def _fin():
        cnt_ref[0, 0] = acc_ref[0]
        sum_ref[0, 0] = acc_ref[1]


@functools.partial(jax.jit, static_argnames=("interpret",))
def _loss_call(preds, labels, interpret=False):
    grid = (_B, _H // _BH)
    loss, cnt, hsum = pl.pallas_call(
        _loss_body,
        grid=grid,
        in_specs=[
            pl.BlockSpec((1, _C, _BH, _W), lambda i, j: (i, 0, j, 0)),
            pl.BlockSpec((1, _BH, _W), lambda i, j: (i, j, 0)),
        ],
        out_specs=[
            pl.BlockSpec((1, _BH, _W), lambda i, j: (i, j, 0)),
            pl.BlockSpec(memory_space=pltpu.SMEM),
            pl.BlockSpec(memory_space=pltpu.SMEM),
        ],
        out_shape=[
            jax.ShapeDtypeStruct((_B, _H, _W), jnp.float32),
            jax.ShapeDtypeStruct((1, 1), jnp.float32),
            jax.ShapeDtypeStruct((1, 1), jnp.float32),
        ],
        scratch_shapes=[pltpu.SMEM((2,), jnp.float32)],
        interpret=interpret,
    )(preds, labels)
    return loss.reshape(-1), cnt[0, 0], hsum[0, 0]


_NT = 16
_PT = _NPIX // _NT
_CH = 8192
_NCH = _PT // _CH
_NB = 2048
_NG = _NB // 16


def _sc_topk_body(loss_hbm, out_hbm, buf, hist_c, m1, m2c, tmp_c, rsum,
                  sabuf, sibuf, res, sh1, sh2c, sh_sa, sh_si):
    cid = lax.axis_index("c")
    sid = lax.axis_index("s")
    lane = lax.iota(jnp.int32, 16)
    base = sid * _PT
    ones_i = jnp.ones((16,), jnp.int32)
    zeros_i = jnp.zeros((16,), jnp.int32)
    zeros_f = jnp.zeros((16,), jnp.float32)

    def _zero_hist(ref):
        @plsc.parallel_loop(0, _NB, unroll=8)
        def zb(g):
            ref[pl.ds(g * 16, 16)] = zeros_i

    _zero_hist(hist_c)

    def chunk1(c, _):
        pltpu.sync_copy(loss_hbm.at[pl.ds(base + c * _CH, _CH)], buf)

        @plsc.parallel_loop(0, _CH // 16, unroll=8)
        def inner(v):
            x = buf[pl.ds(v * 16, 16)]
            k = lax.bitcast_convert_type(x, jnp.int32) & jnp.int32(0x7FFFFFFF)
            b1 = lax.shift_right_logical(k, 20)
            plsc.addupdate_scatter(hist_c, [b1 * 16 + lane], ones_i)
        return 0
    lax.fori_loop(0, _NCH, chunk1, 0)

    @plsc.parallel_loop(0, _NG, unroll=2)
    def lm1(g):
        acc = zeros_i
        for c in range(16):
            acc = acc + plsc.load_gather(hist_c, [g * 256 + lane * 16 + c])
        m1[pl.ds(g * 16, 16)] = acc

    pltpu.sync_copy(m1, sh1.at[sid])
    plsc.subcore_barrier()
    for t in range(_NT):
        pltpu.sync_copy(sh1.at[t], tmp_c)
        if t == 0:
            @plsc.parallel_loop(0, _NG, unroll=4)
            def cp0(g):
                m1[pl.ds(g * 16, 16)] = tmp_c[pl.ds(g * 16, 16)]
        else:
            @plsc.parallel_loop(0, _NG, unroll=4)
            def acc1(g):
                sl = pl.ds(g * 16, 16)
                m1[sl] = m1[sl] + tmp_c[sl]

    T1 = jnp.int32(_NPIX - _NMIN)

    def sc1(g, carry):
        run, cnt = carry
        v = m1[pl.ds(g * 16, 16)]
        cs = plsc.cumsum(v)
        pe = cs + run - v
        pc = plsc.all_reduce_population_count(pe <= T1)
        return run + jnp.max(cs), cnt + jnp.max(pc)
    _, cnt1 = lax.fori_loop(0, _NG, sc1, (jnp.int32(0), jnp.int32(0)))
    b1s = cnt1 - 1
    b1v = jnp.broadcast_to(b1s, (16,))

    def ca(g, acc):
        v = m1[pl.ds(g * 16, 16)]
        binid = g * 16 + lane
        return acc + jnp.sum(jnp.where(binid > b1v, v, zeros_i))
    c_above = lax.fori_loop(0, _NG, ca, jnp.int32(0))

    _zero_hist(hist_c)

    def chunk2(c, sacc):
        pltpu.sync_copy(loss_hbm.at[pl.ds(base + c * _CH, _CH)], buf)

        def inner(v, sacc):
            x = buf[pl.ds(v * 16, 16)]
            k = lax.bitcast_convert_type(x, jnp.int32) & jnp.int32(0x7FFFFFFF)
            b1 = lax.shift_right_logical(k, 20)
            eq = b1 == b1v
            b2 = lax.shift_right_logical(k, 9) & jnp.int32(0x7FF)
            plsc.addupdate_scatter(hist_c, [b2 * 16 + lane], ones_i, mask=eq)
            return sacc + jnp.where(b1 > b1v, x, zeros_f)
        return plsc.parallel_loop(0, _CH // 16, unroll=8, carry=sacc)(inner)
    sa_vec = lax.fori_loop(0, _NCH, chunk2, zeros_f)

    @plsc.parallel_loop(0, _NG, unroll=2)
    def lm2(g):
        acc = zeros_i
        for c in range(16):
            acc = acc + plsc.load_gather(hist_c, [g * 256 + lane * 16 + c])
        m2c[pl.ds(g * 16, 16)] = acc

    pltpu.sync_copy(m2c, sh2c.at[sid])
    plsc.subcore_barrier()
    for t in range(_NT):
        pltpu.sync_copy(sh2c.at[t], tmp_c)
        if t == 0:
            @plsc.parallel_loop(0, _NG, unroll=4)
            def cp2(g):
                m2c[pl.ds(g * 16, 16)] = tmp_c[pl.ds(g * 16, 16)]
        else:
            @plsc.parallel_loop(0, _NG, unroll=4)
            def acc2(g):
                sl = pl.ds(g * 16, 16)
                m2c[sl] = m2c[sl] + tmp_c[sl]

    total2 = jnp.max(plsc.load_gather(m1, [b1v]))
    n2 = jnp.int32(_NMIN) - c_above
    T2 = total2 - n2

    def sc2(g, carry):
        run, cnt = carry
        v = m2c[pl.ds(g * 16, 16)]
        cs = plsc.cumsum(v)
        pe = cs + run - v
        pc = plsc.all_reduce_population_count(pe <= T2)
        return run + jnp.max(cs), cnt + jnp.max(pc)
    _, cnt2 = lax.fori_loop(0, _NG, sc2, (jnp.int32(0), jnp.int32(0)))
    b2s = cnt2 - 1
    b2v = jnp.broadcast_to(b2s, (16,))

    def suf2(g, acc):
        binid = g * 16 + lane
        gt = binid > b2v
        return acc + jnp.sum(jnp.where(gt, m2c[pl.ds(g * 16, 16)], zeros_i))
    c_hi2 = lax.fori_loop(0, _NG, suf2, jnp.int32(0))

    def chunk3(c, sacc):
        pltpu.sync_copy(loss_hbm.at[pl.ds(base + c * _CH, _CH)], buf)

        def inner(v, sacc):
            x = buf[pl.ds(v * 16, 16)]
            k = lax.bitcast_convert_type(x, jnp.int32) & jnp.int32(0x7FFFFFFF)
            b1 = lax.shift_right_logical(k, 20)
            b2 = lax.shift_right_logical(k, 9) & jnp.int32(0x7FF)
            m = (b1 == b1v) & (b2 > b2v)
            return sacc + jnp.where(m, x, zeros_f)
        return plsc.parallel_loop(0, _CH // 16, unroll=8, carry=sacc)(inner)
    si_vec = lax.fori_loop(0, _NCH, chunk3, zeros_f)

    sabuf[...] = sa_vec
    sibuf[...] = si_vec
    pltpu.sync_copy(sabuf, sh_sa.at[pl.ds(sid * 16, 16)])
    pltpu.sync_copy(sibuf, sh_si.at[pl.ds(sid * 16, 16)])
    plsc.subcore_barrier()
    pltpu.sync_copy(sh_sa, rsum)
    s_above = zeros_f
    for t in range(_NT):
        s_above = s_above + rsum[pl.ds(t * 16, 16)]
    pltpu.sync_copy(sh_si, rsum)
    s_in = zeros_f
    for t in range(_NT):
        s_in = s_in + rsum[pl.ds(t * 16, 16)]
    s_hi = jnp.sum(s_above) + jnp.sum(s_in)

    c_hi = c_above + c_hi2
    tbits = jnp.broadcast_to(
        lax.shift_left(b1s, 20) | lax.shift_left(b2s, 9) | jnp.int32(256), (16,))
    t_rep = lax.bitcast_convert_type(tbits, jnp.float32)
    rem = (jnp.int32(_NMIN) - c_hi).astype(jnp.float32)
    res[...] = (s_hi + rem * t_rep) * jnp.float32(1.0 / _NMIN)

    @pl.when((cid == 0) & (sid == 0))
    def _out():
        pltpu.sync_copy(res, out_hbm)


@jax.jit
def _sc_topk_call(loss):
    mesh = plsc.VectorSubcoreMesh(core_axis_name="c", subcore_axis_name="s")
    f = functools.partial(
        pl.kernel,
        out_type=jax.ShapeDtypeStruct((16,), jnp.float32),
        mesh=mesh,
        compiler_params=pltpu.CompilerParams(needs_layout_passes=False),
        scratch_types=[
            pltpu.VMEM((_CH,), jnp.float32),
            pltpu.VMEM((_NB * 16,), jnp.int32),
            pltpu.VMEM((_NB,), jnp.int32),
            pltpu.VMEM((_NB,), jnp.int32),
            pltpu.VMEM((_NB,), jnp.int32),
            pltpu.VMEM((_NT * 16,), jnp.float32),
            pltpu.VMEM((16,), jnp.float32),
            pltpu.VMEM((16,), jnp.float32),
            pltpu.VMEM((16,), jnp.float32),
            pltpu.VMEM_SHARED((_NT, _NB), jnp.int32),
            pltpu.VMEM_SHARED((_NT, _NB), jnp.int32),
            pltpu.VMEM_SHARED((_NT * 16,), jnp.float32),
            pltpu.VMEM_SHARED((_NT * 16,), jnp.float32),
        ],
    )(_sc_topk_body)
    return f(loss)


def kernel(preds, labels):
    loss, n_hard, hard_sum = _loss_call(preds, labels)
    mean_topk = _sc_topk_call(loss)[0]
    mean_hard = hard_sum / n_hard
    return jnp.where(n_hard < jnp.float32(_NMIN), mean_topk, mean_hard)
